# baseline (speedup 1.0000x reference)
import functools

import numpy as np
import jax
import jax.numpy as jnp
from jax import lax
from jax.experimental import pallas as pl
from jax.experimental.pallas import tpu as pltpu
from jax.experimental.pallas import tpu_sc as plsc

VOCAB = 100000
EMBED = 128
BATCH = 4
SEQ = 2048
N = BATCH * SEQ

NUM_CORES = 2
NUM_SUBCORES = 16
NW = NUM_CORES * NUM_SUBCORES
BPW = N // NW
SPB = SEQ // BPW
CH = 32
NCH = BPW // CH
STG = 4 * BPW // NUM_SUBCORES


def _pos_encoding_np() -> np.ndarray:
    pos = np.arange(SEQ, dtype=np.float32)[:, None]
    _2i = np.arange(0, EMBED, 2, dtype=np.float32)
    angle = pos / np.power(10000.0, _2i / EMBED)
    enc = np.zeros((SEQ, EMBED), dtype=np.float32)
    enc[:, 0::2] = np.sin(angle)
    enc[:, 1::2] = np.cos(angle)
    return enc


_POS = _pos_encoding_np()
_POS_DEV = None


def _pos_device():
    global _POS_DEV
    if _POS_DEV is None:
        _POS_DEV = jnp.asarray(_POS, dtype=jnp.bfloat16)
    return _POS_DEV


def _cvt_body(i_ref, o_ref):
    o_ref[...] = i_ref[...].astype(jnp.float32)


_cvt = pl.pallas_call(
    _cvt_body,
    out_shape=jax.ShapeDtypeStruct((SEQ, EMBED), jnp.float32),
    in_specs=[pl.BlockSpec((SEQ // 8, EMBED), lambda i: (i, 0))],
    out_specs=pl.BlockSpec((SEQ // 8, EMBED), lambda i: (i, 0)),
    grid=(8,),
)


_MESH = plsc.VectorSubcoreMesh(core_axis_name="c", subcore_axis_name="s")


@functools.partial(
    pl.kernel,
    out_type=jax.ShapeDtypeStruct((BATCH, SEQ, EMBED), jnp.float32),
    mesh=_MESH,
    scratch_types=[
        pltpu.VMEM((BPW,), jnp.int32),
        pltpu.VMEM((BPW, EMBED), jnp.float32),
        pltpu.VMEM_SHARED((4 * BPW, EMBED), jnp.float32),
        pltpu.SemaphoreType.DMA,
        [pltpu.SemaphoreType.DMA] * NCH,
        [pltpu.SemaphoreType.DMA] * NCH,
        pltpu.SemaphoreType.DMA,
    ],
)
def _emb_kernel(table_hbm, idx_hbm, pos_hbm, out_hbm,
                idx_v, rows_v, pos_sh, sem_i, sems_p, sems_g, sem_w):
    cidx = lax.axis_index("c")
    sidx = lax.axis_index("s")
    wid = sidx * NUM_CORES + cidx
    b = wid // SPB
    s0 = (wid % SPB) * BPW

    idx_cp = pltpu.async_copy(idx_hbm.at[b, pl.ds(s0, BPW)], idx_v, sem_i)

    q = sidx // 4
    src0 = lax.rem(2 * q + cidx, SPB) * BPW + lax.rem(sidx, 4) * STG
    pltpu.sync_copy(pos_hbm.at[pl.ds(src0, STG)], pos_sh.at[pl.ds(sidx * STG, STG)])
    plsc.subcore_barrier()

    p0 = lax.rem(sidx, 4) * BPW
    prefills = []
    for j in range(NCH):
        prefills.append(
            pltpu.async_copy(
                pos_sh.at[pl.ds(p0 + j * CH, CH)],
                rows_v.at[pl.ds(j * CH, CH)],
                sems_p[j],
            )
        )
    idx_cp.wait()

    gathers = []
    for j in range(NCH):
        prefills[j].wait()
        gathers.append(
            pltpu.async_copy(
                table_hbm.at[idx_v.at[pl.ds(j * CH, CH)]],
                rows_v.at[pl.ds(j * CH, CH)],
                sems_g[j],
                add=True,
            )
        )

    outs = []
    for j in range(NCH):
        gathers[j].wait()
        outs.append(
            pltpu.async_copy(
                rows_v.at[pl.ds(j * CH, CH)],
                out_hbm.at[b, pl.ds(s0 + j * CH, CH)],
                sem_w,
            )
        )
    for o in outs:
        o.wait()


@jax.jit
def _impl(x, table, pos):
    return _emb_kernel(table, x, _cvt(pos))


def kernel(x, table):
    return _impl(x, table, _pos_device())


# speedup vs baseline: 1.0835x; 1.0835x over previous
import functools

import numpy as np
import jax
import jax.numpy as jnp
from jax import lax
from jax.experimental import pallas as pl
from jax.experimental.pallas import tpu as pltpu
from jax.experimental.pallas import tpu_sc as plsc

VOCAB = 100000
EMBED = 128
BATCH = 4
SEQ = 2048
N = BATCH * SEQ

NUM_CORES = 2
NUM_SUBCORES = 16
NW = NUM_CORES * NUM_SUBCORES
BPW = N // NW
SPB = SEQ // BPW
CH = 32
NCH = BPW // CH
STG = 4 * BPW // NUM_SUBCORES


def _pos_encoding_np() -> np.ndarray:
    pos = np.arange(SEQ, dtype=np.float32)[:, None]
    _2i = np.arange(0, EMBED, 2, dtype=np.float32)
    angle = pos / np.power(10000.0, _2i / EMBED)
    enc = np.zeros((SEQ, EMBED), dtype=np.float32)
    enc[:, 0::2] = np.sin(angle)
    enc[:, 1::2] = np.cos(angle)
    return enc


_POS = _pos_encoding_np()
_POS_DEV = None


def _pos_device():
    global _POS_DEV
    if _POS_DEV is None:
        _POS_DEV = jnp.asarray(_POS)
    return _POS_DEV


_MESH = plsc.VectorSubcoreMesh(core_axis_name="c", subcore_axis_name="s")


@functools.partial(
    pl.kernel,
    out_type=jax.ShapeDtypeStruct((BATCH, SEQ, EMBED), jnp.float32),
    mesh=_MESH,
    scratch_types=[
        pltpu.VMEM((BPW,), jnp.int32),
        pltpu.VMEM((BPW, EMBED), jnp.float32),
        pltpu.VMEM_SHARED((4 * BPW, EMBED), jnp.float32),
        pltpu.SemaphoreType.DMA,
        [pltpu.SemaphoreType.DMA] * NCH,
        [pltpu.SemaphoreType.DMA] * NCH,
        pltpu.SemaphoreType.DMA,
    ],
)
def _emb_kernel(table_hbm, idx_hbm, pos_hbm, out_hbm,
                idx_v, rows_v, pos_sh, sem_i, sems_p, sems_g, sem_w):
    cidx = lax.axis_index("c")
    sidx = lax.axis_index("s")
    wid = sidx * NUM_CORES + cidx
    b = wid // SPB
    s0 = (wid % SPB) * BPW

    idx_cp = pltpu.async_copy(idx_hbm.at[b, pl.ds(s0, BPW)], idx_v, sem_i)

    q = sidx // 4
    src0 = lax.rem(2 * q + cidx, SPB) * BPW + lax.rem(sidx, 4) * STG
    pltpu.sync_copy(pos_hbm.at[pl.ds(src0, STG)], pos_sh.at[pl.ds(sidx * STG, STG)])
    plsc.subcore_barrier()

    p0 = lax.rem(sidx, 4) * BPW
    prefills = []
    for j in range(NCH):
        prefills.append(
            pltpu.async_copy(
                pos_sh.at[pl.ds(p0 + j * CH, CH)],
                rows_v.at[pl.ds(j * CH, CH)],
                sems_p[j],
            )
        )
    idx_cp.wait()

    gathers = []
    for j in range(NCH):
        prefills[j].wait()
        gathers.append(
            pltpu.async_copy(
                table_hbm.at[idx_v.at[pl.ds(j * CH, CH)]],
                rows_v.at[pl.ds(j * CH, CH)],
                sems_g[j],
                add=True,
            )
        )

    outs = []
    for j in range(NCH):
        gathers[j].wait()
        outs.append(
            pltpu.async_copy(
                rows_v.at[pl.ds(j * CH, CH)],
                out_hbm.at[b, pl.ds(s0 + j * CH, CH)],
                sem_w,
            )
        )
    for o in outs:
        o.wait()


@jax.jit
def _impl(x, table, pos):
    return _emb_kernel(table, x, pos)


def kernel(x, table):
    return _impl(x, table, _pos_device())
